# baseline (speedup 1.0000x reference)
import functools

import jax
import jax.numpy as jnp
from jax import lax
from jax.experimental import pallas as pl
from jax.experimental.pallas import tpu as pltpu
from jax.experimental.pallas import tpu_sc as plsc

NUM_ROWS = 100000
LENGTH = 64
BATCH = 16384

NUM_CORES = 2
NUM_SUBCORES = 16
NUM_WORKERS = NUM_CORES * NUM_SUBCORES
COLS_PER_TILE = LENGTH // NUM_WORKERS
LANES = 16
CHUNK = 1024
NCHUNK = BATCH // CHUNK


def _sc_body(xT_hbm, idx_hbm, bT_hbm, outT_hbm,
             idx_v, idx_sh, col_v, xb0, xb1, ob0, ob1,
             semi, semc, sx0, sx1, so0, so1):
    sid = lax.axis_index("s")
    wid = sid * NUM_CORES + lax.axis_index("c")
    xbufs = (xb0, xb1)
    obufs = (ob0, ob1)
    sxs = (sx0, sx1)
    sos = (so0, so1)

    @pl.when(sid == 0)
    def _():
        pltpu.sync_copy(idx_hbm, idx_sh)

    plsc.subcore_barrier()
    icp = pltpu.async_copy(idx_sh, idx_v, semi)
    icp.wait()

    ocp = [None, None]
    for cc in range(COLS_PER_TILE):
        c = wid * COLS_PER_TILE + cc
        ccp = pltpu.async_copy(bT_hbm.at[c], col_v, semc)
        xcp = [None, None]
        xcp[0] = pltpu.async_copy(
            xT_hbm.at[c, pl.ds(0, CHUNK)], xbufs[0], sxs[0])
        ccp.wait()
        for j in range(NCHUNK):
            cur = j & 1
            nxt = (j + 1) & 1
            if j + 1 < NCHUNK:
                xcp[nxt] = pltpu.async_copy(
                    xT_hbm.at[c, pl.ds((j + 1) * CHUNK, CHUNK)],
                    xbufs[nxt], sxs[nxt])
            xcp[cur].wait()
            if ocp[cur] is not None:
                ocp[cur].wait()
                ocp[cur] = None
            off = j * CHUNK
            x_v = xbufs[cur]
            o_v = obufs[cur]

            @plsc.parallel_loop(0, CHUNK, LANES, unroll=4)
            def do_vec(k):
                iv = idx_v[pl.ds(off + k, LANES)]
                vals = plsc.load_gather(col_v, [iv])
                o_v[pl.ds(k, LANES)] = vals + x_v[pl.ds(k, LANES)]

            ocp[cur] = pltpu.async_copy(
                o_v, outT_hbm.at[c, pl.ds(off, CHUNK)], sos[cur])
    ocp[0].wait()
    ocp[1].wait()


@jax.jit
def _run(xT, batch, bT):
    mesh = plsc.VectorSubcoreMesh(
        core_axis_name="c", subcore_axis_name="s",
        num_cores=NUM_CORES, num_subcores=NUM_SUBCORES,
    )
    return pl.kernel(
        _sc_body,
        out_type=jax.ShapeDtypeStruct((LENGTH, BATCH), jnp.float32),
        mesh=mesh,
        scratch_types=[
            pltpu.VMEM((BATCH,), jnp.int32),
            pltpu.VMEM_SHARED((BATCH,), jnp.int32),
            pltpu.VMEM((NUM_ROWS,), jnp.float32),
            pltpu.VMEM((CHUNK,), jnp.float32),
            pltpu.VMEM((CHUNK,), jnp.float32),
            pltpu.VMEM((CHUNK,), jnp.float32),
            pltpu.VMEM((CHUNK,), jnp.float32),
            pltpu.SemaphoreType.DMA,
            pltpu.SemaphoreType.DMA,
            pltpu.SemaphoreType.DMA,
            pltpu.SemaphoreType.DMA,
            pltpu.SemaphoreType.DMA,
            pltpu.SemaphoreType.DMA,
        ],
        compiler_params=pltpu.CompilerParams(needs_layout_passes=False),
    )(xT, batch, bT)


def kernel(x, batch, b):
    outT = _run(x.T, batch.astype(jnp.int32), b.T)
    return outT.T


# speedup vs baseline: 1.3042x; 1.3042x over previous
import functools

import jax
import jax.numpy as jnp
from jax import lax
from jax.experimental import pallas as pl
from jax.experimental.pallas import tpu as pltpu
from jax.experimental.pallas import tpu_sc as plsc

NUM_ROWS = 100000
LENGTH = 64
BATCH = 16384

NUM_CORES = 2
NUM_SUBCORES = 16
NUM_WORKERS = NUM_CORES * NUM_SUBCORES
COLS_PER_TILE = LENGTH // NUM_WORKERS
LANES = 16
CHUNK = 4096
NCHUNK = BATCH // CHUNK


def _sc_body(xT_hbm, idx_hbm, bT_hbm, outT_hbm,
             idx_sh, col_v, xb0, xb1, ob0, ob1, ib0, ib1,
             semc, sx0, sx1, so0, so1, si0, si1):
    sid = lax.axis_index("s")
    wid = sid * NUM_CORES + lax.axis_index("c")
    xbufs = (xb0, xb1)
    obufs = (ob0, ob1)
    ibufs = (ib0, ib1)
    sxs = (sx0, sx1)
    sos = (so0, so1)
    sis = (si0, si1)

    @pl.when(sid == 0)
    def _():
        pltpu.sync_copy(idx_hbm, idx_sh)

    plsc.subcore_barrier()

    ocp = [None, None]
    for cc in range(COLS_PER_TILE):
        c = wid * COLS_PER_TILE + cc
        ccp = pltpu.async_copy(bT_hbm.at[c], col_v, semc)
        xcp = [None, None]
        icp = [None, None]
        xcp[0] = pltpu.async_copy(
            xT_hbm.at[c, pl.ds(0, CHUNK)], xbufs[0], sxs[0])
        icp[0] = pltpu.async_copy(
            idx_sh.at[pl.ds(0, CHUNK)], ibufs[0], sis[0])
        ccp.wait()
        for j in range(NCHUNK):
            cur = j & 1
            nxt = (j + 1) & 1
            if j + 1 < NCHUNK:
                xcp[nxt] = pltpu.async_copy(
                    xT_hbm.at[c, pl.ds((j + 1) * CHUNK, CHUNK)],
                    xbufs[nxt], sxs[nxt])
                icp[nxt] = pltpu.async_copy(
                    idx_sh.at[pl.ds((j + 1) * CHUNK, CHUNK)],
                    ibufs[nxt], sis[nxt])
            xcp[cur].wait()
            icp[cur].wait()
            if ocp[cur] is not None:
                ocp[cur].wait()
                ocp[cur] = None
            x_v = xbufs[cur]
            o_v = obufs[cur]
            i_v = ibufs[cur]

            @plsc.parallel_loop(0, CHUNK, LANES, unroll=4)
            def do_vec(k):
                iv = i_v[pl.ds(k, LANES)]
                vals = plsc.load_gather(col_v, [iv])
                o_v[pl.ds(k, LANES)] = vals + x_v[pl.ds(k, LANES)]

            ocp[cur] = pltpu.async_copy(
                o_v, outT_hbm.at[c, pl.ds(j * CHUNK, CHUNK)], sos[cur])
    ocp[0].wait()
    ocp[1].wait()


@jax.jit
def _run(xT, batch, bT):
    mesh = plsc.VectorSubcoreMesh(
        core_axis_name="c", subcore_axis_name="s",
        num_cores=NUM_CORES, num_subcores=NUM_SUBCORES,
    )
    return pl.kernel(
        _sc_body,
        out_type=jax.ShapeDtypeStruct((LENGTH, BATCH), jnp.float32),
        mesh=mesh,
        scratch_types=[
            pltpu.VMEM_SHARED((BATCH,), jnp.int32),
            pltpu.VMEM((NUM_ROWS,), jnp.float32),
            pltpu.VMEM((CHUNK,), jnp.float32),
            pltpu.VMEM((CHUNK,), jnp.float32),
            pltpu.VMEM((CHUNK,), jnp.float32),
            pltpu.VMEM((CHUNK,), jnp.float32),
            pltpu.VMEM((CHUNK,), jnp.int32),
            pltpu.VMEM((CHUNK,), jnp.int32),
            pltpu.SemaphoreType.DMA,
            pltpu.SemaphoreType.DMA,
            pltpu.SemaphoreType.DMA,
            pltpu.SemaphoreType.DMA,
            pltpu.SemaphoreType.DMA,
            pltpu.SemaphoreType.DMA,
            pltpu.SemaphoreType.DMA,
        ],
        compiler_params=pltpu.CompilerParams(needs_layout_passes=False),
    )(xT, batch, bT)


def kernel(x, batch, b):
    outT = _run(x.T, batch.astype(jnp.int32), b.T)
    return outT.T
